# baseline (speedup 1.0000x reference)
import functools

import jax
import jax.numpy as jnp
from jax import lax
from jax.experimental import pallas as pl
from jax.experimental.pallas import tpu as pltpu
from jax.experimental.pallas import tpu_sc as plsc

BATCH = 4096
SEQ = 77
SEQP = 80
EMBED = 768
LANES = 16
NCOL = EMBED // LANES
UNITS = ((0, 40), (40, 32))
HEAD = 72
TAILR = SEQ - HEAD
MINI = 8
GRP = 8


def kernel(input_ids, token_table, pos_table):
    info = plsc.get_sparse_core_info()
    nw = info.num_cores * info.num_subcores
    seq_per_w = BATCH // nw

    mesh = plsc.VectorSubcoreMesh(core_axis_name="c", subcore_axis_name="s")

    @functools.partial(
        pl.kernel,
        out_type=jax.ShapeDtypeStruct((BATCH, SEQ, EMBED), jnp.float32),
        mesh=mesh,
        scratch_types=[
            pltpu.VMEM((2 * GRP * SEQP,), jnp.int32),
            pltpu.VMEM((SEQ * EMBED,), jnp.float32),
            pltpu.VMEM((1, UNITS[0][1], EMBED), jnp.float32),
            pltpu.VMEM((1, UNITS[1][1], EMBED), jnp.float32),
            pltpu.VMEM((MINI, EMBED), jnp.float32),
            pltpu.VMEM((1, TAILR, EMBED), jnp.float32),
            pltpu.SemaphoreType.DMA,
            pltpu.SemaphoreType.DMA,
            pltpu.SemaphoreType.DMA,
            pltpu.SemaphoreType.DMA,
            pltpu.SemaphoreType.DMA,
        ],
    )
    def run(ids_hbm, tok_hbm, pos_hbm, out_hbm, ids_v, pos_v, b0, b1,
            mini_v, tail_v, g0, g1, gm, ts, ids_sem):
        wid = lax.axis_index("s") * info.num_cores + lax.axis_index("c")
        base = wid * seq_per_w
        bufs = (b0, b1)
        gsems = (g0, g1)
        pltpu.sync_copy(pos_hbm, pos_v)

        def slot_off(i):
            return lax.rem(lax.div(i, GRP), 2) * (GRP * SEQP)

        def idx(i, u):
            off = slot_off(i) + lax.rem(i, GRP) * SEQP + UNITS[u][0]
            return ids_v.at[pl.ds(off, UNITS[u][1])]

        def idx_mini(i):
            off = slot_off(i) + lax.rem(i, GRP) * SEQP + HEAD
            return ids_v.at[pl.ds(off, MINI)]

        def fire_ids_group(g):
            pltpu.async_copy(
                ids_hbm.at[pl.ds((base + g * GRP) * SEQP, GRP * SEQP)],
                ids_v.at[pl.ds(lax.rem(g, 2) * (GRP * SEQP), GRP * SEQP)],
                ids_sem)

        pltpu.sync_copy(ids_hbm.at[pl.ds(base * SEQP, GRP * SEQP)],
                        ids_v.at[pl.ds(0, GRP * SEQP)])
        for u in range(2):
            pltpu.async_copy(tok_hbm.at[idx(0, u)], bufs[u].at[0], gsems[u])
        pltpu.async_copy(tok_hbm.at[idx_mini(0)], mini_v, gm)

        def seq_body(i, carry):
            seq = base + i
            phase = lax.rem(i, GRP)

            @pl.when((phase == GRP - 2) & (i < seq_per_w - GRP))
            def _():
                fire_ids_group(lax.div(i, GRP) + 1)

            @pl.when((phase == GRP - 1) & (i < seq_per_w - 1))
            def _():
                pltpu.make_async_copy(
                    ids_hbm.at[pl.ds(base * SEQP, GRP * SEQP)],
                    ids_v.at[pl.ds(0, GRP * SEQP)], ids_sem).wait()

            def add_unit(u):
                uoff, ulen = UNITS[u]
                pltpu.make_async_copy(tok_hbm.at[idx(i, u)], bufs[u].at[0],
                                      gsems[u]).wait()

                def row_body(r, c3):
                    pbase = (uoff + r) * EMBED
                    for c in range(NCOL):
                        sl = pl.ds(c * LANES, LANES)
                        bufs[u][0, r, sl] = (
                            bufs[u][0, r, sl]
                            + pos_v[pl.ds(pbase + c * LANES, LANES)])
                    return c3

                lax.fori_loop(0, ulen, row_body, 0)
                return pltpu.async_copy(
                    bufs[u], out_hbm.at[pl.ds(seq, 1), pl.ds(uoff, ulen)],
                    gsems[u])

            st0 = add_unit(0)
            st1 = add_unit(1)
            st0.wait()

            @pl.when(i < seq_per_w - 1)
            def _():
                pltpu.async_copy(tok_hbm.at[idx(i + 1, 0)], bufs[0].at[0],
                                 gsems[0])

            @pl.when(i > 0)
            def _():
                pltpu.make_async_copy(
                    tail_v, out_hbm.at[pl.ds(seq, 1), pl.ds(HEAD, TAILR)],
                    ts).wait()

            pltpu.make_async_copy(tok_hbm.at[idx_mini(i)], mini_v, gm).wait()

            def tail_body(t, c3):
                pbase = (HEAD + t) * EMBED
                for c in range(NCOL):
                    sl = pl.ds(c * LANES, LANES)
                    tail_v[0, t, sl] = (
                        mini_v[t, sl]
                        + pos_v[pl.ds(pbase + c * LANES, LANES)])
                return c3

            lax.fori_loop(0, TAILR, tail_body, 0)
            pltpu.async_copy(tail_v,
                             out_hbm.at[pl.ds(seq, 1), pl.ds(HEAD, TAILR)],
                             ts)
            st1.wait()

            @pl.when(i < seq_per_w - 1)
            def _():
                pltpu.async_copy(tok_hbm.at[idx(i + 1, 1)], bufs[1].at[0],
                                 gsems[1])
                pltpu.async_copy(tok_hbm.at[idx_mini(i + 1)], mini_v, gm)

            return carry

        lax.fori_loop(0, seq_per_w, seq_body, 0)
        pltpu.make_async_copy(
            tail_v,
            out_hbm.at[pl.ds(base + seq_per_w - 1, 1), pl.ds(HEAD, TAILR)],
            ts).wait()

    ids_pad = jnp.pad(input_ids.astype(jnp.int32), ((0, 0), (0, SEQP - SEQ)))
    return run(ids_pad.reshape(-1), token_table, pos_table.reshape(-1))


# speedup vs baseline: 1.0860x; 1.0860x over previous
import functools

import jax
import jax.numpy as jnp
from jax import lax
from jax.experimental import pallas as pl
from jax.experimental.pallas import tpu as pltpu
from jax.experimental.pallas import tpu_sc as plsc

BATCH = 4096
SEQ = 77
SEQP = 80
HEAD = 72
TAIL = 8
EMBED = 768
LANES = 16
GRP = 8


def kernel(input_ids, token_table, pos_table):
    info = plsc.get_sparse_core_info()
    nw = info.num_cores * info.num_subcores
    seq_per_w = BATCH // nw
    n_grp = seq_per_w // GRP

    mesh = plsc.VectorSubcoreMesh(core_axis_name="c", subcore_axis_name="s")

    @functools.partial(
        pl.kernel,
        out_type=jax.ShapeDtypeStruct((BATCH, SEQ, EMBED), jnp.float32),
        mesh=mesh,
        scratch_types=[
            pltpu.VMEM((GRP, SEQP), jnp.int32),
            pltpu.VMEM((SEQ, EMBED), jnp.float32),
            pltpu.VMEM((1, SEQP, EMBED), jnp.float32),
            pltpu.VMEM((1, SEQ - HEAD, EMBED), jnp.float32),
            pltpu.SemaphoreType.DMA,
        ],
    )
    def run(ids_hbm, tok_hbm, pos_hbm, out_hbm, ids_v, pos_v, rows_v,
            tail_v, sem):
        wid = lax.axis_index("s") * info.num_cores + lax.axis_index("c")
        base = wid * seq_per_w
        pltpu.sync_copy(pos_hbm, pos_v)

        def grp_body(g, carry):
            seq0 = base + g * GRP
            pltpu.sync_copy(ids_hbm.at[pl.ds(seq0, GRP)], ids_v)
            for s in range(GRP):
                pltpu.async_copy(tok_hbm.at[ids_v.at[s]], rows_v.at[0],
                                 sem).wait()

                def row_body(r, c3):
                    for c in range(EMBED // LANES):
                        sl = pl.ds(c * LANES, LANES)
                        rows_v[0, r, sl] = rows_v[0, r, sl] + pos_v[r, sl]
                    return c3

                lax.fori_loop(0, HEAD, row_body, 0)

                def tail_body(t, c3):
                    for c in range(EMBED // LANES):
                        sl = pl.ds(c * LANES, LANES)
                        tail_v[0, t, sl] = (rows_v[0, HEAD + t, sl]
                                            + pos_v[HEAD + t, sl])
                    return c3

                lax.fori_loop(0, SEQ - HEAD, tail_body, 0)
                pltpu.sync_copy(
                    rows_v.at[pl.ds(0, 1), pl.ds(0, HEAD)],
                    out_hbm.at[pl.ds(seq0 + s, 1), pl.ds(0, HEAD)])
                pltpu.sync_copy(
                    tail_v,
                    out_hbm.at[pl.ds(seq0 + s, 1), pl.ds(HEAD, SEQ - HEAD)])
            return carry

        lax.fori_loop(0, n_grp, grp_body, 0)

    ids_pad = jnp.pad(input_ids.astype(jnp.int32), ((0, 0), (0, SEQP - SEQ)))
    return run(ids_pad, token_table, pos_table)


# speedup vs baseline: 1.0885x; 1.0023x over previous
import functools

import jax
import jax.numpy as jnp
from jax import lax
from jax.experimental import pallas as pl
from jax.experimental.pallas import tpu as pltpu
from jax.experimental.pallas import tpu_sc as plsc

BATCH = 4096
SEQ = 77
SEQP = 80
HEAD = 72
TAIL = 8
EMBED = 768
LANES = 16
GRP = 8


def kernel(input_ids, token_table, pos_table):
    info = plsc.get_sparse_core_info()
    nw = info.num_cores * info.num_subcores
    seq_per_w = BATCH // nw
    n_grp = seq_per_w // GRP

    mesh = plsc.VectorSubcoreMesh(core_axis_name="c", subcore_axis_name="s")

    @functools.partial(
        pl.kernel,
        out_type=jax.ShapeDtypeStruct((BATCH, SEQ, EMBED), jnp.float32),
        mesh=mesh,
        scratch_types=[
            pltpu.VMEM((GRP, SEQP), jnp.int32),
            pltpu.VMEM((SEQ, EMBED), jnp.float32),
            pltpu.VMEM((1, SEQ, EMBED), jnp.float32),
            pltpu.VMEM((TAIL, EMBED), jnp.float32),
            pltpu.SemaphoreType.DMA,
        ],
    )
    def run(ids_hbm, tok_hbm, pos_hbm, out_hbm, ids_v, pos_v, rows_v,
            tail_v, sem):
        wid = lax.axis_index("s") * info.num_cores + lax.axis_index("c")
        base = wid * seq_per_w
        pltpu.sync_copy(pos_hbm, pos_v)

        def grp_body(g, carry):
            seq0 = base + g * GRP
            pltpu.sync_copy(ids_hbm.at[pl.ds(seq0, GRP)], ids_v)
            for s in range(GRP):
                head_cp = pltpu.async_copy(
                    tok_hbm.at[ids_v.at[s, pl.ds(0, HEAD)]],
                    rows_v.at[0, pl.ds(0, HEAD)], sem)
                tail_cp = pltpu.async_copy(
                    tok_hbm.at[ids_v.at[s, pl.ds(HEAD, TAIL)]], tail_v, sem)
                head_cp.wait()
                tail_cp.wait()

                def row_body(r, c3):
                    for c in range(EMBED // LANES):
                        sl = pl.ds(c * LANES, LANES)
                        rows_v[0, r, sl] = rows_v[0, r, sl] + pos_v[r, sl]
                    return c3

                lax.fori_loop(0, HEAD, row_body, 0)

                def tail_body(t, c3):
                    for c in range(EMBED // LANES):
                        sl = pl.ds(c * LANES, LANES)
                        rows_v[0, HEAD + t, sl] = (tail_v[t, sl]
                                                   + pos_v[HEAD + t, sl])
                    return c3

                lax.fori_loop(0, SEQ - HEAD, tail_body, 0)
                pltpu.sync_copy(rows_v, out_hbm.at[pl.ds(seq0 + s, 1)])
            return carry

        lax.fori_loop(0, n_grp, grp_body, 0)

    ids_pad = jnp.pad(input_ids.astype(jnp.int32), ((0, 0), (0, SEQP - SEQ)))
    return run(ids_pad, token_table, pos_table)
